# baseline (speedup 1.0000x reference)
import jax
import jax.numpy as jnp
from jax import lax
from jax.experimental import pallas as pl
from jax.experimental.pallas import tpu as pltpu
from jax.experimental.pallas import tpu_sc as plsc

N = 10000
C = 128
E = 320000
NLAYERS = 3

NC = 2
NS = 16
NW = NC * NS
CHUNK = 128
NCHUNKS = 80
EPW = NCHUNKS * CHUNK
EPAD = NW * EPW
NPAD = 10240
SUBROWS = 640
SINK0 = 10016


def _adj_body(u_hbm, row_hbm, col_hbm, out_hbm, rowi, coli, buf, acc, g0):
    c = lax.axis_index("c")
    s = lax.axis_index("s")
    wid = c * NS + s

    def zrow(i, _):
        for j in range(C // 16):
            buf[i, pl.ds(j * 16, 16)] = jnp.zeros((16,), jnp.float32)
        return 0

    lax.fori_loop(0, CHUNK, zrow, 0)
    zbase = s * SUBROWS
    for t in range(SUBROWS // CHUNK):
        pltpu.sync_copy(buf, acc.at[pl.ds(zbase + t * CHUNK, CHUNK)])

    pltpu.sync_copy(row_hbm.at[wid], rowi)
    pltpu.sync_copy(col_hbm.at[wid], coli)
    plsc.subcore_barrier()

    def body(j, _):
        pltpu.async_copy(u_hbm.at[rowi.at[j]], buf, g0).wait()
        pltpu.sync_copy(buf, acc.at[coli.at[j]], add=True)
        return 0

    lax.fori_loop(0, NCHUNKS, body, 0)
    plsc.subcore_barrier()

    for t in range(SUBROWS // CHUNK):
        off = zbase + t * CHUNK
        pltpu.sync_copy(acc.at[pl.ds(off, CHUNK)], buf)
        pltpu.sync_copy(buf, out_hbm.at[c, pl.ds(off, CHUNK)])


def _deg_body(row_hbm, out_hbm, ri0, ones, buf, acc, i0):
    c = lax.axis_index("c")
    s = lax.axis_index("s")
    wid = c * NS + s

    for j in range(CHUNK // 16):
        ones[pl.ds(j * 16, 16)] = jnp.ones((16,), jnp.float32)

    def zchunk(i, _):
        buf[pl.ds(i * 16, 16)] = jnp.zeros((16,), jnp.float32)
        return 0

    lax.fori_loop(0, SUBROWS // 16, zchunk, 0)
    pltpu.sync_copy(buf, acc.at[pl.ds(s * SUBROWS, SUBROWS)])
    plsc.subcore_barrier()

    def body(j, _):
        pltpu.async_copy(row_hbm.at[wid, j], ri0, i0).wait()
        pltpu.sync_copy(ones, acc.at[ri0.at[0]], add=True)
        return 0

    lax.fori_loop(0, NCHUNKS, body, 0)
    plsc.subcore_barrier()

    pltpu.sync_copy(acc.at[pl.ds(s * SUBROWS, SUBROWS)], buf)
    pltpu.sync_copy(buf, out_hbm.at[c, pl.ds(s * SUBROWS, SUBROWS)])


def _make_sc_kernels():
    mesh = plsc.VectorSubcoreMesh(core_axis_name="c", subcore_axis_name="s",
                                  num_cores=NC, num_subcores=NS)
    adj = pl.kernel(
        _adj_body,
        out_type=jax.ShapeDtypeStruct((NC, NPAD, C), jnp.float32),
        mesh=mesh,
        scratch_types=[
            pltpu.VMEM((NCHUNKS, CHUNK), jnp.int32),
            pltpu.VMEM((NCHUNKS, CHUNK), jnp.int32),
            pltpu.VMEM((CHUNK, C), jnp.float32),
            pltpu.MemorySpace.VMEM_SHARED((NPAD, C), jnp.float32),
            pltpu.SemaphoreType.DMA,
        ],
        name="sc_adj_accumulate",
    )
    deg = pl.kernel(
        _deg_body,
        out_type=jax.ShapeDtypeStruct((NC, NPAD), jnp.float32),
        mesh=mesh,
        scratch_types=[
            pltpu.VMEM((1, CHUNK), jnp.int32),
            pltpu.VMEM((CHUNK,), jnp.float32),
            pltpu.VMEM((SUBROWS,), jnp.float32),
            pltpu.MemorySpace.VMEM_SHARED((NPAD,), jnp.float32),
            pltpu.SemaphoreType.DMA,
        ],
        name="sc_degree",
    )
    return adj, deg


_adj, _deg = _make_sc_kernels()


BLK = 2000
GRID = N // BLK
_P = lax.Precision.HIGHEST


def _dis_body(degp_ref, dis_ref):
    deg = degp_ref[0] + degp_ref[1]
    dis = jnp.where(deg > 0, lax.rsqrt(deg), 0.0)
    dis_ref[...] = dis[:N, None]


def _in_body(x_ref, w_ref, b_ref, dis_ref, h_ref, u_ref):
    h = jnp.dot(x_ref[...], w_ref[...], preferred_element_type=jnp.float32,
                precision=_P) + b_ref[...]
    h_ref[...] = h
    u_ref[...] = dis_ref[...] * h


def _mid_body(sp_ref, dis_ref, u_ref):
    dis = dis_ref[...]
    u_ref[...] = -(dis * dis) * (sp_ref[0] + sp_ref[1])


def _comb_body(h_ref, s1_ref, s2_ref, dis_ref, w_ref, b_ref, g_ref, bb_ref,
               hn_ref, un_ref):
    tx0 = h_ref[...]
    dis = dis_ref[...]
    tx1 = -dis * (s1_ref[0] + s1_ref[1])
    tx2 = -2.0 * dis * (s2_ref[0] + s2_ref[1]) - tx0
    w = w_ref[...]
    t = (jnp.dot(tx0, w[0], preferred_element_type=jnp.float32, precision=_P)
         + jnp.dot(tx1, w[1], preferred_element_type=jnp.float32, precision=_P)
         + jnp.dot(tx2, w[2], preferred_element_type=jnp.float32, precision=_P)
         + b_ref[...])
    t = jnp.maximum(t, 0.0) + tx0
    m = jnp.mean(t, axis=-1, keepdims=True)
    v = jnp.mean((t - m) ** 2, axis=-1, keepdims=True)
    hn = (t - m) * lax.rsqrt(v + 1e-5) * g_ref[...] + bb_ref[...]
    hn_ref[...] = hn
    un_ref[...] = dis * hn


def _out_body(h_ref, w_ref, b_ref, y_ref):
    y_ref[...] = jnp.dot(h_ref[...], w_ref[...],
                         preferred_element_type=jnp.float32,
                         precision=_P) + b_ref[...]


_rowblk = pl.BlockSpec((BLK, C), lambda i: (i, 0))
_wblk = pl.BlockSpec((C, C), lambda i: (0, 0))
_bblk = pl.BlockSpec((1, C), lambda i: (0, 0))
_pblk = pl.BlockSpec((2, BLK, C), lambda i: (0, i, 0))
_dblk = pl.BlockSpec((BLK, 1), lambda i: (i, 0))

_dis_call = pl.pallas_call(
    _dis_body,
    grid=(1,),
    in_specs=[pl.BlockSpec((2, NPAD), lambda i: (0, 0))],
    out_specs=[pl.BlockSpec((N, 1), lambda i: (0, 0))],
    out_shape=[jax.ShapeDtypeStruct((N, 1), jnp.float32)],
)

_in_call = pl.pallas_call(
    _in_body,
    grid=(GRID,),
    in_specs=[_rowblk, _wblk, _bblk, _dblk],
    out_specs=[_rowblk, _rowblk],
    out_shape=[jax.ShapeDtypeStruct((N, C), jnp.float32),
               jax.ShapeDtypeStruct((N, C), jnp.float32)],
)

_mid_call = pl.pallas_call(
    _mid_body,
    grid=(GRID,),
    in_specs=[_pblk, _dblk],
    out_specs=[_rowblk],
    out_shape=[jax.ShapeDtypeStruct((N, C), jnp.float32)],
)

_comb_call = pl.pallas_call(
    _comb_body,
    grid=(GRID,),
    in_specs=[_rowblk, _pblk, _pblk, _dblk,
              pl.BlockSpec((3, C, C), lambda i: (0, 0, 0)), _bblk, _bblk, _bblk],
    out_specs=[_rowblk, _rowblk],
    out_shape=[jax.ShapeDtypeStruct((N, C), jnp.float32),
               jax.ShapeDtypeStruct((N, C), jnp.float32)],
)

_out_call = pl.pallas_call(
    _out_body,
    grid=(GRID,),
    in_specs=[_rowblk, _wblk, _bblk],
    out_specs=_rowblk,
    out_shape=jax.ShapeDtypeStruct((N, C), jnp.float32),
)


def kernel(x, edge_index, batch, w_in, b_in, cheb_w, cheb_b, ln_g, ln_b,
           w_out, b_out):
    row = edge_index[0]
    col = edge_index[1]
    pad = EPAD - E
    sink = SINK0 + (jnp.arange(pad, dtype=jnp.int32) % 128)
    rowp = jnp.concatenate(
        [row, jnp.zeros((pad,), jnp.int32)]).reshape(NW, NCHUNKS, CHUNK)
    colp = jnp.concatenate([col, sink]).reshape(NW, NCHUNKS, CHUNK)
    rowd = jnp.concatenate([row, sink]).reshape(NW, NCHUNKS, 1, CHUNK)

    degp = _deg(rowd)
    (dis,) = _dis_call(degp)
    h, u = _in_call(x, w_in, b_in.reshape(1, C), dis)
    for i in range(NLAYERS):
        s1 = _adj(u, rowp, colp)
        (u1,) = _mid_call(s1, dis)
        s2 = _adj(u1, rowp, colp)
        h, u = _comb_call(h, s1, s2, dis, cheb_w[i], cheb_b[i].reshape(1, C),
                          ln_g[i].reshape(1, C), ln_b[i].reshape(1, C))
    return _out_call(h, w_out, b_out.reshape(1, C))


# speedup vs baseline: 1.6913x; 1.6913x over previous
import jax
import jax.numpy as jnp
from jax import lax
from jax.experimental import pallas as pl
from jax.experimental.pallas import tpu as pltpu
from jax.experimental.pallas import tpu_sc as plsc

N = 10000
C = 128
E = 320000
NLAYERS = 3

NC = 2
NS = 16
NW = NC * NS
CHUNK = 128
NCHUNKS = 79
EPW = NCHUNKS * CHUNK
EPAD = NW * EPW
NPAD = 10240
SUBROWS = 640
SINK = 10200


def _adj_body(u_hbm, row_hbm, col_hbm, out_hbm, rowi, coli, buf, acc, g0):
    c = lax.axis_index("c")
    s = lax.axis_index("s")
    wid = c * NS + s

    def zrow(i, _):
        for j in range(C // 16):
            buf[i, pl.ds(j * 16, 16)] = jnp.zeros((16,), jnp.float32)
        return 0

    lax.fori_loop(0, CHUNK, zrow, 0)
    zbase = s * SUBROWS
    for t in range(SUBROWS // CHUNK):
        pltpu.sync_copy(buf, acc.at[pl.ds(zbase + t * CHUNK, CHUNK)])

    pltpu.sync_copy(row_hbm.at[wid], rowi)
    pltpu.sync_copy(col_hbm.at[wid], coli)
    plsc.subcore_barrier()

    def body(j, _):
        pltpu.async_copy(u_hbm.at[rowi.at[j]], buf, g0).wait()
        pltpu.sync_copy(buf, acc.at[coli.at[j]], add=True)
        return 0

    lax.fori_loop(0, NCHUNKS, body, 0)
    plsc.subcore_barrier()

    for t in range(SUBROWS // CHUNK):
        off = zbase + t * CHUNK
        pltpu.sync_copy(acc.at[pl.ds(off, CHUNK)], buf)
        pltpu.sync_copy(buf, out_hbm.at[c, pl.ds(off, CHUNK)])


def _deg_body(row_hbm, out_hbm, rowi, ones, buf, acc):
    c = lax.axis_index("c")
    s = lax.axis_index("s")
    wid = c * NS + s

    for j in range(CHUNK // 16):
        ones[pl.ds(j * 16, 16)] = jnp.ones((16,), jnp.float32)

    def zchunk(i, _):
        buf[pl.ds(i * 16, 16)] = jnp.zeros((16,), jnp.float32)
        return 0

    lax.fori_loop(0, SUBROWS // 16, zchunk, 0)
    pltpu.sync_copy(buf, acc.at[pl.ds(s * SUBROWS, SUBROWS)])

    pltpu.sync_copy(row_hbm.at[wid], rowi)
    plsc.subcore_barrier()

    def body(j, _):
        pltpu.sync_copy(ones, acc.at[rowi.at[j]], add=True)
        return 0

    lax.fori_loop(0, NCHUNKS, body, 0)
    plsc.subcore_barrier()

    pltpu.sync_copy(acc.at[pl.ds(s * SUBROWS, SUBROWS)], buf)
    pltpu.sync_copy(buf, out_hbm.at[c, pl.ds(s * SUBROWS, SUBROWS)])


def _make_sc_kernels():
    mesh = plsc.VectorSubcoreMesh(core_axis_name="c", subcore_axis_name="s",
                                  num_cores=NC, num_subcores=NS)
    adj = pl.kernel(
        _adj_body,
        out_type=jax.ShapeDtypeStruct((NC, NPAD, C), jnp.float32),
        mesh=mesh,
        scratch_types=[
            pltpu.VMEM((NCHUNKS, CHUNK), jnp.int32),
            pltpu.VMEM((NCHUNKS, CHUNK), jnp.int32),
            pltpu.VMEM((CHUNK, C), jnp.float32),
            pltpu.MemorySpace.VMEM_SHARED((NPAD, C), jnp.float32),
            pltpu.SemaphoreType.DMA,
        ],
        name="sc_adj_accumulate",
    )
    deg = pl.kernel(
        _deg_body,
        out_type=jax.ShapeDtypeStruct((NC, NPAD), jnp.float32),
        mesh=mesh,
        scratch_types=[
            pltpu.VMEM((NCHUNKS, CHUNK), jnp.int32),
            pltpu.VMEM((CHUNK,), jnp.float32),
            pltpu.VMEM((SUBROWS,), jnp.float32),
            pltpu.MemorySpace.VMEM_SHARED((NPAD,), jnp.float32),
        ],
        name="sc_degree",
    )
    return adj, deg


_adj, _deg = _make_sc_kernels()


BLK = 2000
GRID = N // BLK
_P = lax.Precision.HIGHEST


def _dis_body(degp_ref, dis_ref):
    deg = degp_ref[0] + degp_ref[1]
    dis = jnp.where(deg > 0, lax.rsqrt(deg), 0.0)
    dis_ref[...] = dis[:N, None]


def _in_body(x_ref, w_ref, b_ref, dis_ref, h_ref, u_ref):
    h = jnp.dot(x_ref[...], w_ref[...], preferred_element_type=jnp.float32,
                precision=_P) + b_ref[...]
    h_ref[...] = h
    u_ref[...] = dis_ref[...] * h


def _mid_body(sp_ref, dis_ref, u_ref):
    dis = dis_ref[...]
    u_ref[...] = -(dis * dis) * (sp_ref[0] + sp_ref[1])


def _comb_body(h_ref, s1_ref, s2_ref, dis_ref, w_ref, b_ref, g_ref, bb_ref,
               hn_ref, un_ref):
    tx0 = h_ref[...]
    dis = dis_ref[...]
    tx1 = -dis * (s1_ref[0] + s1_ref[1])
    tx2 = -2.0 * dis * (s2_ref[0] + s2_ref[1]) - tx0
    w = w_ref[...]
    t = (jnp.dot(tx0, w[0], preferred_element_type=jnp.float32, precision=_P)
         + jnp.dot(tx1, w[1], preferred_element_type=jnp.float32, precision=_P)
         + jnp.dot(tx2, w[2], preferred_element_type=jnp.float32, precision=_P)
         + b_ref[...])
    t = jnp.maximum(t, 0.0) + tx0
    m = jnp.mean(t, axis=-1, keepdims=True)
    v = jnp.mean((t - m) ** 2, axis=-1, keepdims=True)
    hn = (t - m) * lax.rsqrt(v + 1e-5) * g_ref[...] + bb_ref[...]
    hn_ref[...] = hn
    un_ref[...] = dis * hn


def _out_body(h_ref, w_ref, b_ref, y_ref):
    y_ref[...] = jnp.dot(h_ref[...], w_ref[...],
                         preferred_element_type=jnp.float32,
                         precision=_P) + b_ref[...]


_rowblk = pl.BlockSpec((BLK, C), lambda i: (i, 0))
_wblk = pl.BlockSpec((C, C), lambda i: (0, 0))
_bblk = pl.BlockSpec((1, C), lambda i: (0, 0))
_pblk = pl.BlockSpec((2, BLK, C), lambda i: (0, i, 0))
_dblk = pl.BlockSpec((BLK, 1), lambda i: (i, 0))

_dis_call = pl.pallas_call(
    _dis_body,
    grid=(1,),
    in_specs=[pl.BlockSpec((2, NPAD), lambda i: (0, 0))],
    out_specs=[pl.BlockSpec((N, 1), lambda i: (0, 0))],
    out_shape=[jax.ShapeDtypeStruct((N, 1), jnp.float32)],
)

_in_call = pl.pallas_call(
    _in_body,
    grid=(GRID,),
    in_specs=[_rowblk, _wblk, _bblk, _dblk],
    out_specs=[_rowblk, _rowblk],
    out_shape=[jax.ShapeDtypeStruct((N, C), jnp.float32),
               jax.ShapeDtypeStruct((N, C), jnp.float32)],
)

_mid_call = pl.pallas_call(
    _mid_body,
    grid=(GRID,),
    in_specs=[_pblk, _dblk],
    out_specs=[_rowblk],
    out_shape=[jax.ShapeDtypeStruct((N, C), jnp.float32)],
)

_comb_call = pl.pallas_call(
    _comb_body,
    grid=(GRID,),
    in_specs=[_rowblk, _pblk, _pblk, _dblk,
              pl.BlockSpec((3, C, C), lambda i: (0, 0, 0)), _bblk, _bblk, _bblk],
    out_specs=[_rowblk, _rowblk],
    out_shape=[jax.ShapeDtypeStruct((N, C), jnp.float32),
               jax.ShapeDtypeStruct((N, C), jnp.float32)],
)

_out_call = pl.pallas_call(
    _out_body,
    grid=(GRID,),
    in_specs=[_rowblk, _wblk, _bblk],
    out_specs=_rowblk,
    out_shape=jax.ShapeDtypeStruct((N, C), jnp.float32),
)


def kernel(x, edge_index, batch, w_in, b_in, cheb_w, cheb_b, ln_g, ln_b,
           w_out, b_out):
    row = edge_index[0]
    col = edge_index[1]
    pad = EPAD - E
    rowp = jnp.concatenate(
        [row, jnp.zeros((pad,), jnp.int32)]).reshape(NW, NCHUNKS, CHUNK)
    colp = jnp.concatenate(
        [col, jnp.full((pad,), SINK, jnp.int32)]).reshape(NW, NCHUNKS, CHUNK)
    rowd = jnp.concatenate(
        [row, jnp.full((pad,), SINK, jnp.int32)]).reshape(NW, NCHUNKS, CHUNK)

    degp = _deg(rowd)
    (dis,) = _dis_call(degp)
    h, u = _in_call(x, w_in, b_in.reshape(1, C), dis)
    for i in range(NLAYERS):
        s1 = _adj(u, rowp, colp)
        (u1,) = _mid_call(s1, dis)
        s2 = _adj(u1, rowp, colp)
        h, u = _comb_call(h, s1, s2, dis, cheb_w[i], cheb_b[i].reshape(1, C),
                          ln_g[i].reshape(1, C), ln_b[i].reshape(1, C))
    return _out_call(h, w_out, b_out.reshape(1, C))
